# baseline (speedup 1.0000x reference)
import functools

import jax
import jax.numpy as jnp
from jax import lax
from jax.experimental import pallas as pl
from jax.experimental.pallas import tpu as pltpu
from jax.experimental.pallas import tpu_sc as plsc

_B, _S, _D = 4, 2048, 768
_NC, _NS = 2, 16
_NW = _NC * _NS
_PPW = _S // _NW
_CH = 32
_LANES = 16

_mesh = plsc.VectorSubcoreMesh(core_axis_name="c", subcore_axis_name="s")


@functools.partial(
    pl.kernel,
    mesh=_mesh,
    out_type=jax.ShapeDtypeStruct((_B * _S, _D), jnp.float32),
    scratch_types=[
        pltpu.VMEM((_CH,), jnp.int32),
        pltpu.VMEM((_PPW, _D), jnp.float32),
        pltpu.VMEM((_CH, _D), jnp.float32),
    ],
)
def _emb_kernel(tok_hbm, ids_hbm, pos_hbm, out_hbm, idx_v, pos_v, tok_v):
    wid = lax.axis_index("s") * _NC + lax.axis_index("c")
    p0 = wid * _PPW
    pltpu.sync_copy(pos_hbm.at[pl.ds(p0, _PPW)], pos_v)

    @pl.loop(0, _B)
    def _batch(b):
        @pl.loop(0, _PPW // _CH)
        def _chunk(k):
            off = b * _S + p0 + k * _CH
            pltpu.sync_copy(ids_hbm.at[pl.ds(off, _CH)], idx_v)
            pltpu.sync_copy(tok_hbm.at[idx_v], tok_v)

            @pl.loop(0, _CH)
            def _row(j):
                @pl.loop(0, _D, step=_LANES)
                def _col(cc):
                    tok_v[j, pl.ds(cc, _LANES)] = (
                        tok_v[j, pl.ds(cc, _LANES)]
                        + pos_v[k * _CH + j, pl.ds(cc, _LANES)]
                    )

            pltpu.sync_copy(tok_v, out_hbm.at[pl.ds(off, _CH)])


def kernel(inputs, W_tok, W_pos):
    ids = inputs.reshape(-1).astype(jnp.int32)
    out = _emb_kernel(W_tok, ids, W_pos)
    return out.reshape(_B, _S, _D)


# speedup vs baseline: 1.6875x; 1.6875x over previous
import functools

import jax
import jax.numpy as jnp
from jax import lax
from jax.experimental import pallas as pl
from jax.experimental.pallas import tpu as pltpu
from jax.experimental.pallas import tpu_sc as plsc

_B, _S, _D = 4, 2048, 768
_NC, _NS = 2, 16
_NW = _NC * _NS
_PPW = _S // _NW
_CH = 32
_NCHUNK = _B * _PPW // _CH
_LANES = 16

_mesh = plsc.VectorSubcoreMesh(core_axis_name="c", subcore_axis_name="s")


@functools.partial(
    pl.kernel,
    mesh=_mesh,
    out_type=jax.ShapeDtypeStruct((_B * _S, _D), jnp.float32),
    scratch_types=[
        pltpu.VMEM((_B, _PPW), jnp.int32),
        pltpu.VMEM((_PPW, _D), jnp.float32),
        pltpu.VMEM((2, _CH, _D), jnp.float32),
        pltpu.SemaphoreType.DMA,
        pltpu.SemaphoreType.DMA,
        pltpu.SemaphoreType.DMA,
        pltpu.SemaphoreType.DMA,
        pltpu.SemaphoreType.DMA,
    ],
)
def _emb_kernel(tok_hbm, ids_hbm, pos_hbm, out_hbm, idx_v, pos_v, tok_v,
                isem, gsem0, gsem1, ssem0, ssem1):
    wid = lax.axis_index("s") * _NC + lax.axis_index("c")
    p0 = wid * _PPW
    gsem = (gsem0, gsem1)
    ssem = (ssem0, ssem1)

    idx_copies = [
        pltpu.async_copy(ids_hbm.at[pl.ds(b * _S + p0, _PPW)], idx_v.at[b], isem)
        for b in range(_B)
    ]
    pltpu.sync_copy(pos_hbm.at[pl.ds(p0, _PPW)], pos_v)
    for h in idx_copies:
        h.wait()

    gh = [None] * _NCHUNK
    sh = [None] * _NCHUNK

    def start_gather(i):
        b, k = divmod(i, _PPW // _CH)
        buf = i % 2
        gh[i] = pltpu.async_copy(
            tok_hbm.at[idx_v.at[b, pl.ds(k * _CH, _CH)]],
            tok_v.at[buf], gsem[buf])

    start_gather(0)
    for i in range(_NCHUNK):
        buf = i % 2
        if i + 1 < _NCHUNK:
            if i - 1 >= 0:
                sh[i - 1].wait()
            start_gather(i + 1)
        gh[i].wait()
        b, k = divmod(i, _PPW // _CH)

        @pl.loop(0, _CH)
        def _row(j, k=k, buf=buf):
            for cc in range(0, _D, _LANES):
                plsc.addupdate(
                    tok_v.at[buf, j, pl.ds(cc, _LANES)],
                    pos_v[k * _CH + j, pl.ds(cc, _LANES)])

        off = b * _S + p0 + k * _CH
        sh[i] = pltpu.async_copy(tok_v.at[buf], out_hbm.at[pl.ds(off, _CH)],
                                 ssem[buf])
    sh[_NCHUNK - 2].wait()
    sh[_NCHUNK - 1].wait()


def kernel(inputs, W_tok, W_pos):
    ids = inputs.reshape(-1).astype(jnp.int32)
    out = _emb_kernel(W_tok, ids, W_pos)
    return out.reshape(_B, _S, _D)
